# Initial kernel scaffold; baseline (speedup 1.0000x reference)
import jax
import jax.numpy as jnp
from jax.experimental import pallas as pl


def kernel(x_gene, x_cell, ei_gg, ei_cc, ei_cg, ei_gc, ea_cg, ea_gc, sage_gg_Wl, sage_gg_bl, sage_gg_Wr, sage_cc_Wl, sage_cc_bl, sage_cc_Wr, nn_W1, nn_b1, nn_W2, nn_b2, nn_cg_root, nn_cg_bias, nn_gc_root, nn_gc_bias):
    raise NotImplementedError("write your pallas kernel here")


# speedup vs baseline: 3.8476x; 3.8476x over previous
import functools

import jax
import jax.numpy as jnp
from jax import lax
from jax.experimental import pallas as pl
from jax.experimental.pallas import tpu as pltpu
from jax.experimental.pallas import tpu_sc as plsc

N = 10000
D_IN = 3
H = 64
E = 320000
L = 16
NC, NS = 2, 16
NW = NC * NS
ROW = 128
RPT = 80
EP = NW * RPT * ROW
NROWS3 = EP // ROW
CH = 8
NACC = 10112
ZW = 96
CPT = ZW // NS


def _mesh():
    return plsc.VectorSubcoreMesh(core_axis_name="c", subcore_axis_name="s",
                                  num_cores=NC, num_subcores=NS)


def _z_body(x_hbm, src3, dst3, ea3, wtab, btab, ctab, z_out,
            xt, srcb, dstb, eab, wrow_b, brow_b, crow_b, acc):
    cid = lax.axis_index("c")
    sid = lax.axis_index("s")

    pltpu.sync_copy(x_hbm, xt)
    pltpu.sync_copy(wtab.at[sid], wrow_b)
    pltpu.sync_copy(btab.at[sid], brow_b)
    pltpu.sync_copy(ctab.at[sid], crow_b)

    zeros16 = jnp.zeros((L,), jnp.float32)

    def _zr(i, c):
        acc[pl.ds(i * L, L)] = zeros16
        return c
    lax.fori_loop(0, NACC * CPT // L, _zr, 0)

    wrow = wrow_b[pl.ds(0, L)]
    brow = brow_b[pl.ds(0, L)]
    crow = crow_b[pl.ds(0, L)]
    wj = [wrow[j] for j in range(CPT)]
    bj = [brow[j] for j in range(CPT)]
    is0 = [crow[j] == 0 for j in range(CPT)]
    is1 = [crow[j] == 1 for j in range(CPT)]

    half = NROWS3 // NC
    wbase = cid * half

    def _chunk(ci, c):
        rb = wbase + ci * CH
        pltpu.sync_copy(src3.at[pl.ds(rb, CH)], srcb)
        pltpu.sync_copy(dst3.at[pl.ds(rb, CH)], dstb)
        pltpu.sync_copy(ea3.at[pl.ds(rb, CH)], eab)

        def _row(j, cc):
            def _grp(g, gc):
                base = g * L
                s16 = srcb[j, pl.ds(base, L)] * 3
                a16 = eab[j, pl.ds(base, L)]
                d16 = dstb[j, pl.ds(base, L)] * CPT
                x0 = plsc.load_gather(xt, [s16])
                x1 = plsc.load_gather(xt, [s16 + 1])
                x2 = plsc.load_gather(xt, [s16 + 2])
                for q in range(CPT):
                    h = jnp.maximum(a16 * wj[q] + bj[q], 0.0)
                    xc = jnp.where(is0[q], x0, jnp.where(is1[q], x1, x2))
                    plsc.addupdate_scatter(acc, [d16 + q], xc * h)
                return gc
            lax.fori_loop(0, ROW // L, _grp, 0)
            return cc
        lax.fori_loop(0, CH, _row, 0)
        return c
    lax.fori_loop(0, half // CH, _chunk, 0)

    pltpu.sync_copy(acc, z_out.at[cid, sid])


@functools.cache
def _z_fn():
    return pl.kernel(
        _z_body,
        out_type=jax.ShapeDtypeStruct((NC, NS, NACC * CPT), jnp.float32),
        mesh=_mesh(),
        scratch_types=[
            pltpu.VMEM((N * D_IN,), jnp.float32),
            pltpu.VMEM((CH, ROW), jnp.int32),
            pltpu.VMEM((CH, ROW), jnp.int32),
            pltpu.VMEM((CH, ROW), jnp.float32),
            pltpu.VMEM((L,), jnp.float32),
            pltpu.VMEM((L,), jnp.float32),
            pltpu.VMEM((L,), jnp.int32),
            pltpu.VMEM((NACC * CPT,), jnp.float32),
        ],
        compiler_params=pltpu.CompilerParams(needs_layout_passes=False),
        name="hetero_sc_z",
    )


def _stats_one(xt, src3, dst3, srcb, dstb, acc, out, wid, wbase):
    zeros16 = jnp.zeros((L,), jnp.float32)
    ones16 = jnp.ones((L,), jnp.float32)

    def _zr(i, c):
        acc[pl.ds(i * L, L)] = zeros16
        return c
    lax.fori_loop(0, NACC * 4 // L, _zr, 0)

    def _chunk(ci, c):
        rb = wbase + ci * CH
        pltpu.sync_copy(src3.at[pl.ds(rb, CH)], srcb)
        pltpu.sync_copy(dst3.at[pl.ds(rb, CH)], dstb)

        def _row(j, cc):
            def _grp(g, gc):
                base = g * L
                s16 = srcb[j, pl.ds(base, L)] * 3
                d16 = dstb[j, pl.ds(base, L)] * 4
                x0 = plsc.load_gather(xt, [s16])
                x1 = plsc.load_gather(xt, [s16 + 1])
                x2 = plsc.load_gather(xt, [s16 + 2])
                plsc.addupdate_scatter(acc, [d16], x0)
                plsc.addupdate_scatter(acc, [d16 + 1], x1)
                plsc.addupdate_scatter(acc, [d16 + 2], x2)
                plsc.addupdate_scatter(acc, [d16 + 3], ones16)
                return gc
            lax.fori_loop(0, ROW // L, _grp, 0)
            return cc
        lax.fori_loop(0, CH, _row, 0)
        return c
    lax.fori_loop(0, RPT // CH, _chunk, 0)
    pltpu.sync_copy(acc, out.at[wid])


def _s_body(xg_hbm, xc_hbm, nsrc_g, ndst_g, ssrc_g, sdst_g,
            nsrc_c, ndst_c, ssrc_c, sdst_c,
            xng_out, xsg_out, xnc_out, xsc_out,
            xg, xc, srcb, dstb, acc):
    cid = lax.axis_index("c")
    sid = lax.axis_index("s")
    wid = sid * NC + cid
    wbase = wid * RPT

    pltpu.sync_copy(xg_hbm, xg)
    pltpu.sync_copy(xc_hbm, xc)

    _stats_one(xc, nsrc_g, ndst_g, srcb, dstb, acc, xng_out, wid, wbase)
    _stats_one(xg, ssrc_g, sdst_g, srcb, dstb, acc, xsg_out, wid, wbase)
    _stats_one(xg, nsrc_c, ndst_c, srcb, dstb, acc, xnc_out, wid, wbase)
    _stats_one(xc, ssrc_c, sdst_c, srcb, dstb, acc, xsc_out, wid, wbase)


@functools.cache
def _s_fn():
    xout = jax.ShapeDtypeStruct((NW, NACC * 4), jnp.float32)
    return pl.kernel(
        _s_body,
        out_type=(xout,) * 4,
        mesh=_mesh(),
        scratch_types=[
            pltpu.VMEM((N * D_IN,), jnp.float32),
            pltpu.VMEM((N * D_IN,), jnp.float32),
            pltpu.VMEM((CH, ROW), jnp.int32),
            pltpu.VMEM((CH, ROW), jnp.int32),
            pltpu.VMEM((NACC * 4,), jnp.float32),
        ],
        compiler_params=pltpu.CompilerParams(needs_layout_passes=False),
        name="hetero_sc_stats",
    )


def _pad_edges(ei, ea=None):
    npad = EP - E
    pad_dst = N + (jnp.arange(npad, dtype=jnp.int32) % (NACC - N))
    src = jnp.concatenate([ei[0], jnp.zeros((npad,), jnp.int32)])
    dst = jnp.concatenate([ei[1], pad_dst])
    src3 = src.reshape(NROWS3, ROW)
    dst3 = dst.reshape(NROWS3, ROW)
    if ea is None:
        return src3, dst3, None
    ea3 = jnp.concatenate([ea.reshape(-1), jnp.zeros((npad,), jnp.float32)])
    return src3, dst3, ea3.reshape(NROWS3, ROW)


BR = 2000


def _merge_body(x_ref, z_ref, xn_ref, xs_ref, wx_ref, wl_ref, m_ref, b2_ref,
                bias_ref, o_ref):
    z = z_ref[0] + z_ref[1]
    sel = (lax.broadcasted_iota(jnp.int32, (NW * 4, 4), 0) % 4
           == lax.broadcasted_iota(jnp.int32, (NW * 4, 4), 1)
           ).astype(jnp.float32)
    xn = jnp.dot(xn_ref[...], sel, preferred_element_type=jnp.float32)
    xs = jnp.dot(xs_ref[...], sel, preferred_element_type=jnp.float32)
    cn = jnp.maximum(xn[:, 3:4], 1.0)
    cs = jnp.maximum(xs[:, 3:4], 1.0)
    x = x_ref[...]
    out = jnp.dot(x, wx_ref[...], preferred_element_type=jnp.float32)
    out += jnp.dot(xs[:, 0:3] / cs, wl_ref[...],
                   preferred_element_type=jnp.float32)
    out += jnp.dot(z / cn, m_ref[...], preferred_element_type=jnp.float32)
    out += jnp.dot(xn[:, 0:3] / cn, b2_ref[...],
                   preferred_element_type=jnp.float32)
    o_ref[...] = out + bias_ref[...]


def _merge(x_dst, z2, xn2, xs2, wx, wl, m, b2r, bias):
    full = lambda shape: pl.BlockSpec(shape, lambda i: (0,) * len(shape))
    return pl.pallas_call(
        _merge_body,
        grid=(N // BR,),
        in_specs=[
            pl.BlockSpec((BR, D_IN), lambda i: (i, 0)),
            pl.BlockSpec((NC, BR, ZW), lambda i: (0, i, 0)),
            pl.BlockSpec((BR, NW * 4), lambda i: (i, 0)),
            pl.BlockSpec((BR, NW * 4), lambda i: (i, 0)),
            full((D_IN, H)),
            full((D_IN, H)),
            full((ZW, H)),
            full((D_IN, H)),
            full((1, H)),
        ],
        out_specs=pl.BlockSpec((BR, H), lambda i: (i, 0)),
        out_shape=jax.ShapeDtypeStruct((N, H), jnp.float32),
        name="hetero_tc_merge",
    )(x_dst, z2, xn2, xs2, wx, wl, m, b2r, bias)


def kernel(x_gene, x_cell, ei_gg, ei_cc, ei_cg, ei_gc, ea_cg, ea_gc,
           sage_gg_Wl, sage_gg_bl, sage_gg_Wr,
           sage_cc_Wl, sage_cc_bl, sage_cc_Wr,
           nn_W1, nn_b1, nn_W2, nn_b2,
           nn_cg_root, nn_cg_bias, nn_gc_root, nn_gc_bias):
    xg_flat = x_gene.reshape(-1)
    xc_flat = x_cell.reshape(-1)
    w1 = nn_W1.reshape(-1)
    b1 = nn_b1.reshape(-1)

    cols = jnp.arange(NS * CPT, dtype=jnp.int32)
    kv = cols % 32
    cv = cols // 32
    pad = jnp.zeros((NS, L - CPT), jnp.float32)
    wtab = jnp.concatenate([w1[kv].reshape(NS, CPT), pad], axis=1)
    btab = jnp.concatenate([b1[kv].reshape(NS, CPT), pad], axis=1)
    ctab = jnp.concatenate(
        [cv.reshape(NS, CPT), jnp.zeros((NS, L - CPT), jnp.int32)], axis=1)

    nsrc_g, ndst_g, nea_g = _pad_edges(ei_cg, ea_cg)
    ssrc_g, sdst_g, _ = _pad_edges(ei_gg)
    nsrc_c, ndst_c, nea_c = _pad_edges(ei_gc, ea_gc)
    ssrc_c, sdst_c, _ = _pad_edges(ei_cc)

    z_fn = _z_fn()
    zr_g = z_fn(xc_flat, nsrc_g, ndst_g, nea_g, wtab, btab, ctab)
    zr_c = z_fn(xg_flat, nsrc_c, ndst_c, nea_c, wtab, btab, ctab)
    xn_g, xs_g, xn_c, xs_c = _s_fn()(
        xg_flat, xc_flat, nsrc_g, ndst_g, ssrc_g, sdst_g,
        nsrc_c, ndst_c, ssrc_c, sdst_c)

    z_g = zr_g.reshape(NC, NS, NACC, CPT).transpose(0, 2, 1, 3) \
              .reshape(NC, NACC, ZW)
    z_c = zr_c.reshape(NC, NS, NACC, CPT).transpose(0, 2, 1, 3) \
              .reshape(NC, NACC, ZW)
    def _interleave(a):
        return a.reshape(NW, NACC, 4).transpose(1, 0, 2).reshape(NACC, NW * 4)

    xn_g = _interleave(xn_g)
    xs_g = _interleave(xs_g)
    xn_c = _interleave(xn_c)
    xs_c = _interleave(xs_c)

    m = 0.5 * nn_W2.reshape(32, D_IN, H).transpose(1, 0, 2).reshape(ZW, H)
    b2r = 0.5 * nn_b2.reshape(D_IN, H)

    out_gene = _merge(
        x_gene, z_g, xn_g, xs_g,
        0.5 * (sage_gg_Wr + nn_cg_root), 0.5 * sage_gg_Wl, m, b2r,
        (0.5 * (sage_gg_bl + nn_cg_bias)).reshape(1, H))
    out_cell = _merge(
        x_cell, z_c, xn_c, xs_c,
        0.5 * (sage_cc_Wr + nn_gc_root), 0.5 * sage_cc_Wl, m, b2r,
        (0.5 * (sage_cc_bl + nn_gc_bias)).reshape(1, H))
    return (out_gene, out_cell)


# speedup vs baseline: 5.5402x; 1.4399x over previous
import functools

import jax
import jax.numpy as jnp
from jax import lax
from jax.experimental import pallas as pl
from jax.experimental.pallas import tpu as pltpu
from jax.experimental.pallas import tpu_sc as plsc

N = 10000
D_IN = 3
H = 64
E = 320000
L = 16
NC, NS = 2, 16
NW = NC * NS
ROW = 128
RPT = 80
EP = NW * RPT * ROW
NROWS3 = EP // ROW
CH = 8
CHZ = 32
NACC = 10112
ZW = 96
CPT = ZW // NS


def _mesh():
    return plsc.VectorSubcoreMesh(core_axis_name="c", subcore_axis_name="s",
                                  num_cores=NC, num_subcores=NS)


def _z_body(x_hbm, src3, dst3, ea3, wtab, btab, z_out,
            xt, srcb, dstb, eab, wrow_b, brow_b, acc):
    cid = lax.axis_index("c")
    sid = lax.axis_index("s")

    pltpu.sync_copy(x_hbm, xt)
    pltpu.sync_copy(wtab.at[sid], wrow_b)
    pltpu.sync_copy(btab.at[sid], brow_b)

    zeros16 = jnp.zeros((L,), jnp.float32)

    def _zr(i, c):
        acc[pl.ds(i * L, L)] = zeros16
        return c
    lax.fori_loop(0, NACC * CPT // L, _zr, 0)

    wrow = wrow_b[pl.ds(0, L)]
    brow = brow_b[pl.ds(0, L)]
    w0, w1 = wrow[0], wrow[1]
    b0, b1 = brow[0], brow[1]

    half = NROWS3 // NC
    wbase = cid * half

    def _chunk(ci, c):
        rb = wbase + ci * CHZ
        pltpu.sync_copy(src3.at[pl.ds(rb, CHZ)], srcb)
        pltpu.sync_copy(dst3.at[pl.ds(rb, CHZ)], dstb)
        pltpu.sync_copy(ea3.at[pl.ds(rb, CHZ)], eab)

        def _row(j, cc):
            def _grp(g, gc):
                base = g * L
                s16 = srcb[j, pl.ds(base, L)] * 3
                a16 = eab[j, pl.ds(base, L)]
                d16 = dstb[j, pl.ds(base, L)] * CPT
                x0 = plsc.load_gather(xt, [s16])
                x1 = plsc.load_gather(xt, [s16 + 1])
                x2 = plsc.load_gather(xt, [s16 + 2])
                h0 = jnp.maximum(a16 * w0 + b0, 0.0)
                h1 = jnp.maximum(a16 * w1 + b1, 0.0)
                plsc.addupdate_scatter(acc, [d16], x0 * h0)
                plsc.addupdate_scatter(acc, [d16 + 1], x1 * h0)
                plsc.addupdate_scatter(acc, [d16 + 2], x2 * h0)
                plsc.addupdate_scatter(acc, [d16 + 3], x0 * h1)
                plsc.addupdate_scatter(acc, [d16 + 4], x1 * h1)
                plsc.addupdate_scatter(acc, [d16 + 5], x2 * h1)
                return gc
            lax.fori_loop(0, ROW // L, _grp, 0)
            return cc
        lax.fori_loop(0, CHZ, _row, 0)
        return c
    lax.fori_loop(0, half // CHZ, _chunk, 0)

    pltpu.sync_copy(acc, z_out.at[cid, sid])


@functools.cache
def _z_fn():
    return pl.kernel(
        _z_body,
        out_type=jax.ShapeDtypeStruct((NC, NS, NACC * CPT), jnp.float32),
        mesh=_mesh(),
        scratch_types=[
            pltpu.VMEM((N * D_IN,), jnp.float32),
            pltpu.VMEM((CHZ, ROW), jnp.int32),
            pltpu.VMEM((CHZ, ROW), jnp.int32),
            pltpu.VMEM((CHZ, ROW), jnp.float32),
            pltpu.VMEM((L,), jnp.float32),
            pltpu.VMEM((L,), jnp.float32),
            pltpu.VMEM((NACC * CPT,), jnp.float32),
        ],
        compiler_params=pltpu.CompilerParams(needs_layout_passes=False),
        name="hetero_sc_z",
    )


def _stats_one(xt, src3, dst3, srcb, dstb, acc, out, wid, wbase):
    zeros16 = jnp.zeros((L,), jnp.float32)
    ones16 = jnp.ones((L,), jnp.float32)

    def _zr(i, c):
        acc[pl.ds(i * L, L)] = zeros16
        return c
    lax.fori_loop(0, NACC * 4 // L, _zr, 0)

    def _chunk(ci, c):
        rb = wbase + ci * CH
        pltpu.sync_copy(src3.at[pl.ds(rb, CH)], srcb)
        pltpu.sync_copy(dst3.at[pl.ds(rb, CH)], dstb)

        def _row(j, cc):
            def _grp(g, gc):
                base = g * L
                s16 = srcb[j, pl.ds(base, L)] * 3
                d16 = dstb[j, pl.ds(base, L)] * 4
                x0 = plsc.load_gather(xt, [s16])
                x1 = plsc.load_gather(xt, [s16 + 1])
                x2 = plsc.load_gather(xt, [s16 + 2])
                plsc.addupdate_scatter(acc, [d16], x0)
                plsc.addupdate_scatter(acc, [d16 + 1], x1)
                plsc.addupdate_scatter(acc, [d16 + 2], x2)
                plsc.addupdate_scatter(acc, [d16 + 3], ones16)
                return gc
            lax.fori_loop(0, ROW // L, _grp, 0)
            return cc
        lax.fori_loop(0, CH, _row, 0)
        return c
    lax.fori_loop(0, RPT // CH, _chunk, 0)
    pltpu.sync_copy(acc, out.at[wid])


def _s_body(xg_hbm, xc_hbm, nsrc_g, ndst_g, ssrc_g, sdst_g,
            nsrc_c, ndst_c, ssrc_c, sdst_c,
            xng_out, xsg_out, xnc_out, xsc_out,
            xg, xc, srcb, dstb, acc):
    cid = lax.axis_index("c")
    sid = lax.axis_index("s")
    wid = sid * NC + cid
    wbase = wid * RPT

    pltpu.sync_copy(xg_hbm, xg)
    pltpu.sync_copy(xc_hbm, xc)

    _stats_one(xc, nsrc_g, ndst_g, srcb, dstb, acc, xng_out, wid, wbase)
    _stats_one(xg, ssrc_g, sdst_g, srcb, dstb, acc, xsg_out, wid, wbase)
    _stats_one(xg, nsrc_c, ndst_c, srcb, dstb, acc, xnc_out, wid, wbase)
    _stats_one(xc, ssrc_c, sdst_c, srcb, dstb, acc, xsc_out, wid, wbase)


@functools.cache
def _s_fn():
    xout = jax.ShapeDtypeStruct((NW, NACC * 4), jnp.float32)
    return pl.kernel(
        _s_body,
        out_type=(xout,) * 4,
        mesh=_mesh(),
        scratch_types=[
            pltpu.VMEM((N * D_IN,), jnp.float32),
            pltpu.VMEM((N * D_IN,), jnp.float32),
            pltpu.VMEM((CH, ROW), jnp.int32),
            pltpu.VMEM((CH, ROW), jnp.int32),
            pltpu.VMEM((NACC * 4,), jnp.float32),
        ],
        compiler_params=pltpu.CompilerParams(needs_layout_passes=False),
        name="hetero_sc_stats",
    )


def _pad_edges(ei, ea=None):
    npad = EP - E
    pad_dst = N + (jnp.arange(npad, dtype=jnp.int32) % (NACC - N))
    src = jnp.concatenate([ei[0], jnp.zeros((npad,), jnp.int32)])
    dst = jnp.concatenate([ei[1], pad_dst])
    src3 = src.reshape(NROWS3, ROW)
    dst3 = dst.reshape(NROWS3, ROW)
    if ea is None:
        return src3, dst3, None
    ea3 = jnp.concatenate([ea.reshape(-1), jnp.zeros((npad,), jnp.float32)])
    return src3, dst3, ea3.reshape(NROWS3, ROW)


BR = 2000


def _merge_body(x_ref, z_ref, xn_ref, xs_ref, wx_ref, wl_ref, m_ref, b2_ref,
                bias_ref, o_ref):
    z = z_ref[0] + z_ref[1]
    sel = (lax.broadcasted_iota(jnp.int32, (NW * 4, 4), 0) % 4
           == lax.broadcasted_iota(jnp.int32, (NW * 4, 4), 1)
           ).astype(jnp.float32)
    xn = jnp.dot(xn_ref[...], sel, preferred_element_type=jnp.float32)
    xs = jnp.dot(xs_ref[...], sel, preferred_element_type=jnp.float32)
    cn = jnp.maximum(xn[:, 3:4], 1.0)
    cs = jnp.maximum(xs[:, 3:4], 1.0)
    x = x_ref[...]
    out = jnp.dot(x, wx_ref[...], preferred_element_type=jnp.float32)
    out += jnp.dot(xs[:, 0:3] / cs, wl_ref[...],
                   preferred_element_type=jnp.float32)
    out += jnp.dot(z / cn, m_ref[...], preferred_element_type=jnp.float32)
    out += jnp.dot(xn[:, 0:3] / cn, b2_ref[...],
                   preferred_element_type=jnp.float32)
    o_ref[...] = out + bias_ref[...]


def _merge(x_dst, z2, xn2, xs2, wx, wl, m, b2r, bias):
    full = lambda shape: pl.BlockSpec(shape, lambda i: (0,) * len(shape))
    return pl.pallas_call(
        _merge_body,
        grid=(N // BR,),
        in_specs=[
            pl.BlockSpec((BR, D_IN), lambda i: (i, 0)),
            pl.BlockSpec((NC, BR, ZW), lambda i: (0, i, 0)),
            pl.BlockSpec((BR, NW * 4), lambda i: (i, 0)),
            pl.BlockSpec((BR, NW * 4), lambda i: (i, 0)),
            full((D_IN, H)),
            full((D_IN, H)),
            full((ZW, H)),
            full((D_IN, H)),
            full((1, H)),
        ],
        out_specs=pl.BlockSpec((BR, H), lambda i: (i, 0)),
        out_shape=jax.ShapeDtypeStruct((N, H), jnp.float32),
        name="hetero_tc_merge",
    )(x_dst, z2, xn2, xs2, wx, wl, m, b2r, bias)


def kernel(x_gene, x_cell, ei_gg, ei_cc, ei_cg, ei_gc, ea_cg, ea_gc,
           sage_gg_Wl, sage_gg_bl, sage_gg_Wr,
           sage_cc_Wl, sage_cc_bl, sage_cc_Wr,
           nn_W1, nn_b1, nn_W2, nn_b2,
           nn_cg_root, nn_cg_bias, nn_gc_root, nn_gc_bias):
    xg_flat = x_gene.reshape(-1)
    xc_flat = x_cell.reshape(-1)
    w1 = nn_W1.reshape(-1)
    b1 = nn_b1.reshape(-1)

    pad = jnp.zeros((NS, L - 2), jnp.float32)
    wtab = jnp.concatenate([w1.reshape(NS, 2), pad], axis=1)
    btab = jnp.concatenate([b1.reshape(NS, 2), pad], axis=1)

    nsrc_g, ndst_g, nea_g = _pad_edges(ei_cg, ea_cg)
    ssrc_g, sdst_g, _ = _pad_edges(ei_gg)
    nsrc_c, ndst_c, nea_c = _pad_edges(ei_gc, ea_gc)
    ssrc_c, sdst_c, _ = _pad_edges(ei_cc)

    z_fn = _z_fn()
    zr_g = z_fn(xc_flat, nsrc_g, ndst_g, nea_g, wtab, btab)
    zr_c = z_fn(xg_flat, nsrc_c, ndst_c, nea_c, wtab, btab)
    xn_g, xs_g, xn_c, xs_c = _s_fn()(
        xg_flat, xc_flat, nsrc_g, ndst_g, ssrc_g, sdst_g,
        nsrc_c, ndst_c, ssrc_c, sdst_c)

    def _asm(zr):
        return zr.reshape(NC, NS, NACC, 2, D_IN).transpose(0, 2, 4, 1, 3) \
                 .reshape(NC, NACC, ZW)

    z_g = _asm(zr_g)
    z_c = _asm(zr_c)
    def _interleave(a):
        return a.reshape(NW, NACC, 4).transpose(1, 0, 2).reshape(NACC, NW * 4)

    xn_g = _interleave(xn_g)
    xs_g = _interleave(xs_g)
    xn_c = _interleave(xn_c)
    xs_c = _interleave(xs_c)

    m = 0.5 * nn_W2.reshape(32, D_IN, H).transpose(1, 0, 2).reshape(ZW, H)
    b2r = 0.5 * nn_b2.reshape(D_IN, H)

    out_gene = _merge(
        x_gene, z_g, xn_g, xs_g,
        0.5 * (sage_gg_Wr + nn_cg_root), 0.5 * sage_gg_Wl, m, b2r,
        (0.5 * (sage_gg_bl + nn_cg_bias)).reshape(1, H))
    out_cell = _merge(
        x_cell, z_c, xn_c, xs_c,
        0.5 * (sage_cc_Wr + nn_gc_root), 0.5 * sage_cc_Wl, m, b2r,
        (0.5 * (sage_cc_bl + nn_gc_bias)).reshape(1, H))
    return (out_gene, out_cell)
